# Initial kernel scaffold; baseline (speedup 1.0000x reference)
import jax
import jax.numpy as jnp
from jax.experimental import pallas as pl


def kernel(points, W_rel, b_rel, W_dist, b_dist, W_dens, b_dens, W_out, b_out):
    raise NotImplementedError("write your pallas kernel here")


# speedup vs baseline: 48.8738x; 48.8738x over previous
import functools

import jax
import jax.numpy as jnp
from jax.experimental import pallas as pl
from jax.experimental.pallas import tpu as pltpu

B, N, DIM = 16, 2048, 3
SUB = 128
EMBED = 3 * SUB
RB = 256

_INF = float('inf')


def _body(pts_ref, ptsT_ref, wrel_ref, brel_ref, wdist_ref, bdist_ref,
          wdens_ref, bdens_ref, wout_ref, bout_ref, out_ref):
    rb = pl.program_id(1)

    pts_blk = pts_ref[0]
    ptsT = ptsT_ref[0]

    csum = jnp.sum(ptsT, axis=1)
    centroid = (csum / jnp.float32(N))[None, :]
    rel = pts_blk - centroid

    rel_f = jax.lax.dot_general(rel, wrel_ref[...],
                                (((1,), (0,)), ((), ())),
                                preferred_element_type=jnp.float32)
    rel_f = rel_f + brel_ref[...]

    cdist = jnp.sqrt(jnp.sum(rel * rel, axis=1, keepdims=True))
    dist_f = cdist * wdist_ref[...] + bdist_ref[...]

    x2r = jnp.sum(pts_blk * pts_blk, axis=1, keepdims=True)
    x2c = jnp.sum(ptsT * ptsT, axis=0, keepdims=True)
    g = jax.lax.dot_general(pts_blk, ptsT,
                            (((1,), (0,)), ((), ())),
                            preferred_element_type=jnp.float32)
    d2 = jnp.maximum(x2r + x2c - 2.0 * g, 0.0)

    row_ids = rb * RB + jax.lax.broadcasted_iota(jnp.int32, (RB, 1), 0)
    col_ids = jax.lax.broadcasted_iota(jnp.int32, (1, N), 1)
    d2 = jnp.where(row_ids == col_ids, _INF, d2)

    m1 = jnp.min(d2, axis=1, keepdims=True)
    le1 = d2 <= m1
    c1 = jnp.sum(le1.astype(jnp.float32), axis=1, keepdims=True)
    d2b = jnp.where(le1, _INF, d2)
    m2 = jnp.min(d2b, axis=1, keepdims=True)
    le2 = d2b <= m2
    c2 = jnp.sum(le2.astype(jnp.float32), axis=1, keepdims=True)
    d2c = jnp.where(le2, _INF, d2b)
    m3 = jnp.min(d2c, axis=1, keepdims=True)

    three = jnp.float32(3.0)
    n1 = jnp.minimum(c1, three)
    n2 = jnp.minimum(c2, three - n1)
    n3 = three - n1 - n2
    s1 = jnp.sqrt(m1) * n1
    s2 = jnp.where(n2 > 0, jnp.sqrt(m2), 0.0) * n2
    s3 = jnp.where(n3 > 0, jnp.sqrt(m3), 0.0) * n3
    density = (s1 + s2 + s3) / three

    dens_f = density * wdens_ref[...] + bdens_ref[...]

    feat = jnp.concatenate([rel_f, dist_f, dens_f], axis=1)
    out = jax.lax.dot_general(feat, wout_ref[...],
                              (((1,), (0,)), ((), ())),
                              preferred_element_type=jnp.float32)
    out_ref[0] = out + bout_ref[...]


@jax.jit
def kernel(points, W_rel, b_rel, W_dist, b_dist, W_dens, b_dens, W_out, b_out):
    pointsT = jnp.transpose(points, (0, 2, 1))
    grid = (B, N // RB)
    out = pl.pallas_call(
        _body,
        grid=grid,
        in_specs=[
            pl.BlockSpec((1, RB, DIM), lambda b, r: (b, r, 0)),
            pl.BlockSpec((1, DIM, N), lambda b, r: (b, 0, 0)),
            pl.BlockSpec((DIM, SUB), lambda b, r: (0, 0)),
            pl.BlockSpec((1, SUB), lambda b, r: (0, 0)),
            pl.BlockSpec((1, SUB), lambda b, r: (0, 0)),
            pl.BlockSpec((1, SUB), lambda b, r: (0, 0)),
            pl.BlockSpec((1, SUB), lambda b, r: (0, 0)),
            pl.BlockSpec((1, SUB), lambda b, r: (0, 0)),
            pl.BlockSpec((EMBED, EMBED), lambda b, r: (0, 0)),
            pl.BlockSpec((1, EMBED), lambda b, r: (0, 0)),
        ],
        out_specs=pl.BlockSpec((1, RB, EMBED), lambda b, r: (b, r, 0)),
        out_shape=jax.ShapeDtypeStruct((B, N, EMBED), jnp.float32),
        compiler_params=pltpu.CompilerParams(
            dimension_semantics=("parallel", "arbitrary"),
        ),
    )(points, pointsT,
      W_rel, b_rel[None, :], W_dist, b_dist[None, :],
      W_dens, b_dens[None, :], W_out, b_out[None, :])
    return out
